# baseline (speedup 1.0000x reference)
import functools

import jax
import jax.numpy as jnp
from jax import lax
from jax.experimental import pallas as pl
from jax.experimental.pallas import tpu as pltpu
from jax.experimental.pallas import tpu_sc as plsc

_N = 10000
_NPAD = 10240
_E = 320000
_H = 16
_NBR = 2500
_BPW = 80
_NC = 2
_NS = 16

_mesh = plsc.VectorSubcoreMesh(core_axis_name="c", subcore_axis_name="s")


def _load_edges_start(eidx_hbm, vals_hbm, src_v, dst_v, vals_v, wid, sems):
    base = wid * 78 + jnp.minimum(wid, 4)
    pltpu.async_copy(eidx_hbm.at[1, pl.ds(base, 78)],
                     src_v.at[pl.ds(0, 78)], sems[0])
    pltpu.async_copy(eidx_hbm.at[0, pl.ds(base, 78)],
                     dst_v.at[pl.ds(0, 78)], sems[1])
    pltpu.async_copy(vals_hbm.at[pl.ds(base, 78)],
                     vals_v.at[pl.ds(0, 78)], sems[2])

    @pl.when(wid < 4)
    def _extra():
        pltpu.async_copy(eidx_hbm.at[1, base + 78], src_v.at[78], sems[0])
        pltpu.async_copy(eidx_hbm.at[0, base + 78], dst_v.at[78], sems[1])
        pltpu.async_copy(vals_hbm.at[base + 78], vals_v.at[78], sems[2])

    zi16 = jnp.zeros((16,), jnp.int32)
    zf16 = jnp.zeros((16,), jnp.float32)
    for c in range(8):
        sl = pl.ds(c * 16, 16)
        src_v[79, sl] = zi16
        dst_v[79, sl] = zi16
        vals_v[79, sl] = zf16

    @pl.when(wid >= 4)
    def _z78():
        for c in range(8):
            sl = pl.ds(c * 16, 16)
            src_v[78, sl] = zi16
            dst_v[78, sl] = zi16
            vals_v[78, sl] = zf16
    return base


def _load_edges_wait(eidx_hbm, vals_hbm, src_v, dst_v, vals_v, wid, sems,
                     base):
    pltpu.make_async_copy(eidx_hbm.at[1, pl.ds(base, 78)],
                          src_v.at[pl.ds(0, 78)], sems[0]).wait()
    pltpu.make_async_copy(eidx_hbm.at[0, pl.ds(base, 78)],
                          dst_v.at[pl.ds(0, 78)], sems[1]).wait()
    pltpu.make_async_copy(vals_hbm.at[pl.ds(base, 78)],
                          vals_v.at[pl.ds(0, 78)], sems[2]).wait()

    @pl.when(wid < 4)
    def _extra():
        pltpu.make_async_copy(eidx_hbm.at[1, base + 78],
                              src_v.at[78], sems[0]).wait()
        pltpu.make_async_copy(eidx_hbm.at[0, base + 78],
                              dst_v.at[78], sems[1]).wait()
        pltpu.make_async_copy(vals_hbm.at[base + 78],
                              vals_v.at[78], sems[2]).wait()


def _mm1_body(x_ref, w_ref, o_ref):
    o_ref[...] = jnp.dot(x_ref[...], w_ref[...],
                         preferred_element_type=jnp.float32)


def _mid_body(p_ref, b1_ref, w2_ref, o_ref):
    m = p_ref[0] + p_ref[1]
    h = jnp.maximum(m + b1_ref[...], 0.0)
    o_ref[...] = jnp.dot(h, w2_ref[...], preferred_element_type=jnp.float32)


def _fin_body(p_ref, b2_ref, o_ref):
    o_ref[...] = jax.nn.sigmoid(jnp.sum(p_ref[...], axis=0) + b2_ref[...])


@functools.partial(
    pl.kernel,
    out_type=jax.ShapeDtypeStruct((_NC, _NPAD, _H), jnp.float32),
    mesh=_mesh,
    scratch_types=[
        pltpu.VMEM((_BPW, 128), jnp.int32),
        pltpu.VMEM((_BPW, 128), jnp.int32),
        pltpu.VMEM((_BPW, 128), jnp.float32),
        pltpu.VMEM((128, _H), jnp.float32),
        pltpu.VMEM((128, _H), jnp.float32),
        pltpu.VMEM((128, _H), jnp.float32),
        pltpu.VMEM((128, _H), jnp.float32),
        pltpu.VMEM((640, _H), jnp.float32),
        pltpu.VMEM_SHARED((_NPAD, _H), jnp.float32),
        pltpu.SemaphoreType.DMA,
        pltpu.SemaphoreType.DMA,
        pltpu.SemaphoreType.DMA,
        pltpu.SemaphoreType.DMA,
        pltpu.SemaphoreType.DMA,
        pltpu.SemaphoreType.DMA,
        pltpu.SemaphoreType.DMA,
        pltpu.SemaphoreType.DMA,
    ],
    compiler_params=pltpu.CompilerParams(use_tc_tiling_on_sc=False,
                                         needs_layout_passes=False),
)
def _spmm1(t1_hbm, eidx_hbm, vals_hbm, out_hbm,
           src_v, dst_v, vals_v, r0, r1, r2, r3, zeros_v, acc_sh,
           g0, g1, g2, g3, s0, s1, s2, s3):
    rows = (r0, r1, r2, r3)
    gsems = (g0, g1, g2, g3)
    ssems = (s0, s1, s2, s3)
    cid = lax.axis_index("c")
    sid = lax.axis_index("s")
    wid = sid * _NC + cid

    ebase = _load_edges_start(eidx_hbm, vals_hbm, src_v, dst_v, vals_v,
                              wid, (s0, s1, s2))

    zv = jnp.zeros((_H,), jnp.float32)

    @pl.loop(0, 640, unroll=8)
    def _zero(i):
        zeros_v[i, :] = zv

    pltpu.sync_copy(zeros_v, acc_sh.at[pl.ds(sid * 640, 640)])
    _load_edges_wait(eidx_hbm, vals_hbm, src_v, dst_v, vals_v, wid,
                     (s0, s1, s2), ebase)

    plsc.subcore_barrier()

    def _compute(rbuf, b):
        for g in range(8):
            vv = vals_v[b, pl.ds(g * 16, 16)]
            for j in range(16):
                e = g * 16 + j
                bj = jnp.broadcast_to(vv[j], (16,))
                rbuf[e, :] = rbuf[e, :] * bj

    for k in range(4):
        pltpu.async_copy(t1_hbm.at[src_v.at[k]], rows[k], gsems[k])

    @pl.loop(0, 20)
    def _quad(q):
        b0 = q * 4
        for k in range(4):
            b = b0 + k
            pltpu.make_async_copy(t1_hbm.at[src_v.at[b]],
                                  rows[k], gsems[k]).wait()
            _compute(rows[k], b)
            pltpu.async_copy(rows[k], acc_sh.at[dst_v.at[b]], ssems[k],
                             add=True)

        @pl.when(q < 19)
        def _prefetch():
            for k in range(4):
                bn = b0 + 4 + k
                pltpu.make_async_copy(rows[k], acc_sh.at[dst_v.at[bn]],
                                      ssems[k]).wait()
                pltpu.async_copy(t1_hbm.at[src_v.at[bn]], rows[k], gsems[k])

    for k in range(4):
        pltpu.make_async_copy(rows[k], acc_sh.at[dst_v.at[76 + k]],
                              ssems[k]).wait()

    plsc.subcore_barrier()
    pltpu.sync_copy(acc_sh.at[pl.ds(sid * 640, 640)],
                    out_hbm.at[cid, pl.ds(sid * 640, 640)])


@functools.partial(
    pl.kernel,
    out_type=jax.ShapeDtypeStruct((_NC, _NS, 640, _H), jnp.float32),
    mesh=_mesh,
    scratch_types=[
        pltpu.VMEM((_NPAD,), jnp.float32),
        pltpu.VMEM((640, _H), jnp.float32),
        pltpu.VMEM((_BPW, 128), jnp.int32),
        pltpu.VMEM((_BPW, 128), jnp.int32),
        pltpu.VMEM((_BPW, 128), jnp.float32),
        pltpu.SemaphoreType.DMA,
        pltpu.SemaphoreType.DMA,
        pltpu.SemaphoreType.DMA,
        pltpu.SemaphoreType.DMA,
    ],
    compiler_params=pltpu.CompilerParams(use_tc_tiling_on_sc=False,
                                         needs_layout_passes=False),
)
def _spmm2(t2_hbm, eidx_hbm, vals_hbm, out_hbm,
           t2_v, acc_v, src_v, dst_v, vals_v, m0, m1, m2, m3):
    cid = lax.axis_index("c")
    sid = lax.axis_index("s")
    wid = sid * _NC + cid

    pltpu.async_copy(t2_hbm, t2_v, m3)
    ebase = _load_edges_start(eidx_hbm, vals_hbm, src_v, dst_v, vals_v,
                              wid, (m0, m1, m2))

    zv = jnp.zeros((_H,), jnp.float32)

    @pl.loop(0, 640, unroll=8)
    def _zero(i):
        acc_v[i, :] = zv

    pltpu.make_async_copy(t2_hbm, t2_v, m3).wait()
    _load_edges_wait(eidx_hbm, vals_hbm, src_v, dst_v, vals_v, wid,
                     (m0, m1, m2), ebase)

    @pl.loop(0, _BPW)
    def _batch(b):
        for g in range(8):
            sl = pl.ds(g * 16, 16)
            sidx = src_v[b, sl]
            didx = dst_v[b, sl]
            vv = vals_v[b, sl]
            gathered = plsc.load_gather(t2_v, [sidx])
            contrib = gathered * vv
            plsc.addupdate_scatter(acc_v, [didx >> 4, didx & 15], contrib)

    pltpu.sync_copy(acc_v, out_hbm.at[cid, sid])


def kernel(x, edge_index, edge_vals, W1, b1, W2, b2):
    eidx3 = edge_index.reshape(2, _NBR, 128)
    vals2 = edge_vals.reshape(_NBR, 128)

    t1 = pl.pallas_call(
        _mm1_body,
        out_shape=jax.ShapeDtypeStruct((_N, _H), jnp.float32),
    )(x, W1)

    p1 = _spmm1(t1, eidx3, vals2)

    t2 = pl.pallas_call(
        _mid_body,
        out_shape=jax.ShapeDtypeStruct((_NPAD, 1), jnp.float32),
    )(p1, b1.reshape(1, _H), W2)

    p2 = _spmm2(t2.reshape(_NPAD), eidx3, vals2)

    outp = pl.pallas_call(
        _fin_body,
        out_shape=jax.ShapeDtypeStruct((80, 128), jnp.float32),
    )(p2.reshape(_NC * _NS, 80, 128), b2.reshape(1, 1))

    return outp.reshape(_NPAD)[:_N].reshape(_N, 1)


# speedup vs baseline: 1.2057x; 1.2057x over previous
import functools

import jax
import jax.numpy as jnp
from jax import lax
from jax.experimental import pallas as pl
from jax.experimental.pallas import tpu as pltpu
from jax.experimental.pallas import tpu_sc as plsc

_N = 10000
_NPAD = 10240
_E = 320000
_H = 16
_NBR = 2500
_BPW = 80
_NC = 2
_NS = 16

_mesh = plsc.VectorSubcoreMesh(core_axis_name="c", subcore_axis_name="s")


def _load_edges_start(eidx_hbm, vals_hbm, src_v, dst_v, vals_v, wid, sems):
    base = wid * 78 + jnp.minimum(wid, 4)
    pltpu.async_copy(eidx_hbm.at[1, pl.ds(base, 78)],
                     src_v.at[pl.ds(0, 78)], sems[0])
    pltpu.async_copy(eidx_hbm.at[0, pl.ds(base, 78)],
                     dst_v.at[pl.ds(0, 78)], sems[1])
    pltpu.async_copy(vals_hbm.at[pl.ds(base, 78)],
                     vals_v.at[pl.ds(0, 78)], sems[2])

    @pl.when(wid < 4)
    def _extra():
        pltpu.async_copy(eidx_hbm.at[1, base + 78], src_v.at[78], sems[0])
        pltpu.async_copy(eidx_hbm.at[0, base + 78], dst_v.at[78], sems[1])
        pltpu.async_copy(vals_hbm.at[base + 78], vals_v.at[78], sems[2])

    zi16 = jnp.zeros((16,), jnp.int32)
    zf16 = jnp.zeros((16,), jnp.float32)
    for c in range(8):
        sl = pl.ds(c * 16, 16)
        src_v[79, sl] = zi16
        dst_v[79, sl] = zi16
        vals_v[79, sl] = zf16

    @pl.when(wid >= 4)
    def _z78():
        for c in range(8):
            sl = pl.ds(c * 16, 16)
            src_v[78, sl] = zi16
            dst_v[78, sl] = zi16
            vals_v[78, sl] = zf16
    return base


def _load_edges_wait(eidx_hbm, vals_hbm, src_v, dst_v, vals_v, wid, sems,
                     base):
    pltpu.make_async_copy(eidx_hbm.at[1, pl.ds(base, 78)],
                          src_v.at[pl.ds(0, 78)], sems[0]).wait()
    pltpu.make_async_copy(eidx_hbm.at[0, pl.ds(base, 78)],
                          dst_v.at[pl.ds(0, 78)], sems[1]).wait()
    pltpu.make_async_copy(vals_hbm.at[pl.ds(base, 78)],
                          vals_v.at[pl.ds(0, 78)], sems[2]).wait()

    @pl.when(wid < 4)
    def _extra():
        pltpu.make_async_copy(eidx_hbm.at[1, base + 78],
                              src_v.at[78], sems[0]).wait()
        pltpu.make_async_copy(eidx_hbm.at[0, base + 78],
                              dst_v.at[78], sems[1]).wait()
        pltpu.make_async_copy(vals_hbm.at[base + 78],
                              vals_v.at[78], sems[2]).wait()


def _mm1_body(x_ref, w_ref, o_ref):
    o_ref[...] = jnp.dot(x_ref[...], w_ref[...],
                         preferred_element_type=jnp.float32)


def _mid_body(p_ref, b1_ref, w2_ref, o_ref):
    m = p_ref[0] + p_ref[1]
    h = jnp.maximum(m + b1_ref[...], 0.0)
    o_ref[...] = jnp.dot(h, w2_ref[...], preferred_element_type=jnp.float32)


def _fin_body(p_ref, b2_ref, o_ref):
    o_ref[...] = jax.nn.sigmoid(jnp.sum(p_ref[...], axis=0) + b2_ref[...])


@functools.partial(
    pl.kernel,
    out_type=jax.ShapeDtypeStruct((_NC, _NPAD, _H), jnp.float32),
    mesh=_mesh,
    scratch_types=[
        pltpu.VMEM((_BPW, 128), jnp.int32),
        pltpu.VMEM((_BPW, 128), jnp.int32),
        pltpu.VMEM((_BPW, 128), jnp.float32),
        pltpu.VMEM((128, _H), jnp.float32),
        pltpu.VMEM((128, _H), jnp.float32),
        pltpu.VMEM((128, _H), jnp.float32),
        pltpu.VMEM((128, _H), jnp.float32),
        pltpu.VMEM((640, _H), jnp.float32),
        pltpu.VMEM_SHARED((_NPAD, _H), jnp.float32),
        pltpu.VMEM_SHARED((_NPAD, _H), jnp.float32),
        pltpu.SemaphoreType.DMA,
        pltpu.SemaphoreType.DMA,
        pltpu.SemaphoreType.DMA,
        pltpu.SemaphoreType.DMA,
        pltpu.SemaphoreType.DMA,
        pltpu.SemaphoreType.DMA,
        pltpu.SemaphoreType.DMA,
        pltpu.SemaphoreType.DMA,
    ],
    compiler_params=pltpu.CompilerParams(use_tc_tiling_on_sc=False,
                                         needs_layout_passes=False),
)
def _spmm1(t1_hbm, eidx_hbm, vals_hbm, out_hbm,
           src_v, dst_v, vals_v, r0, r1, r2, r3, zeros_v, acc_sh, t1_sh,
           g0, g1, g2, g3, s0, s1, s2, s3):
    rows = (r0, r1, r2, r3)
    gsems = (g0, g1, g2, g3)
    ssems = (s0, s1, s2, s3)
    cid = lax.axis_index("c")
    sid = lax.axis_index("s")
    wid = sid * _NC + cid

    pltpu.async_copy(t1_hbm.at[pl.ds(sid * 625, 625)],
                     t1_sh.at[pl.ds(sid * 625, 625)], s3)
    ebase = _load_edges_start(eidx_hbm, vals_hbm, src_v, dst_v, vals_v,
                              wid, (s0, s1, s2))

    zv = jnp.zeros((_H,), jnp.float32)

    @pl.loop(0, 640, unroll=8)
    def _zero(i):
        zeros_v[i, :] = zv

    pltpu.sync_copy(zeros_v, acc_sh.at[pl.ds(sid * 640, 640)])
    pltpu.make_async_copy(t1_hbm.at[pl.ds(sid * 625, 625)],
                          t1_sh.at[pl.ds(sid * 625, 625)], s3).wait()
    _load_edges_wait(eidx_hbm, vals_hbm, src_v, dst_v, vals_v, wid,
                     (s0, s1, s2), ebase)

    plsc.subcore_barrier()

    def _compute(rbuf, b):
        for g in range(8):
            vv = vals_v[b, pl.ds(g * 16, 16)]
            for j in range(16):
                e = g * 16 + j
                bj = jnp.broadcast_to(vv[j], (16,))
                rbuf[e, :] = rbuf[e, :] * bj

    gsrc = (t1_sh, t1_sh, t1_sh, t1_hbm)
    for k in range(4):
        pltpu.async_copy(gsrc[k].at[src_v.at[k]], rows[k], gsems[k])

    @pl.loop(0, 20)
    def _quad(q):
        b0 = q * 4
        for k in range(4):
            b = b0 + k
            pltpu.make_async_copy(t1_hbm.at[src_v.at[b]],
                                  rows[k], gsems[k]).wait()
            _compute(rows[k], b)
            pltpu.async_copy(rows[k], acc_sh.at[dst_v.at[b]], ssems[k],
                             add=True)

        @pl.when(q < 19)
        def _prefetch():
            for k in range(4):
                bn = b0 + 4 + k
                pltpu.make_async_copy(rows[k], acc_sh.at[dst_v.at[bn]],
                                      ssems[k]).wait()
                pltpu.async_copy(gsrc[k].at[src_v.at[bn]], rows[k],
                                 gsems[k])

    for k in range(4):
        pltpu.make_async_copy(rows[k], acc_sh.at[dst_v.at[76 + k]],
                              ssems[k]).wait()

    plsc.subcore_barrier()
    pltpu.sync_copy(acc_sh.at[pl.ds(sid * 640, 640)],
                    out_hbm.at[cid, pl.ds(sid * 640, 640)])


@functools.partial(
    pl.kernel,
    out_type=jax.ShapeDtypeStruct((_NC, _NS, 640, _H), jnp.float32),
    mesh=_mesh,
    scratch_types=[
        pltpu.VMEM((_NPAD,), jnp.float32),
        pltpu.VMEM((640, _H), jnp.float32),
        pltpu.VMEM((_BPW, 128), jnp.int32),
        pltpu.VMEM((_BPW, 128), jnp.int32),
        pltpu.VMEM((_BPW, 128), jnp.float32),
        pltpu.SemaphoreType.DMA,
        pltpu.SemaphoreType.DMA,
        pltpu.SemaphoreType.DMA,
        pltpu.SemaphoreType.DMA,
    ],
    compiler_params=pltpu.CompilerParams(use_tc_tiling_on_sc=False,
                                         needs_layout_passes=False),
)
def _spmm2(t2_hbm, eidx_hbm, vals_hbm, out_hbm,
           t2_v, acc_v, src_v, dst_v, vals_v, m0, m1, m2, m3):
    cid = lax.axis_index("c")
    sid = lax.axis_index("s")
    wid = sid * _NC + cid

    pltpu.async_copy(t2_hbm, t2_v, m3)
    ebase = _load_edges_start(eidx_hbm, vals_hbm, src_v, dst_v, vals_v,
                              wid, (m0, m1, m2))

    zv = jnp.zeros((_H,), jnp.float32)

    @pl.loop(0, 640, unroll=8)
    def _zero(i):
        acc_v[i, :] = zv

    pltpu.make_async_copy(t2_hbm, t2_v, m3).wait()
    _load_edges_wait(eidx_hbm, vals_hbm, src_v, dst_v, vals_v, wid,
                     (m0, m1, m2), ebase)

    @pl.loop(0, _BPW, unroll=2)
    def _batch(b):
        for g in range(8):
            sl = pl.ds(g * 16, 16)
            sidx = src_v[b, sl]
            didx = dst_v[b, sl]
            vv = vals_v[b, sl]
            gathered = plsc.load_gather(t2_v, [sidx])
            contrib = gathered * vv
            plsc.addupdate_scatter(acc_v, [didx >> 4, didx & 15], contrib)

    pltpu.sync_copy(acc_v, out_hbm.at[cid, sid])


def kernel(x, edge_index, edge_vals, W1, b1, W2, b2):
    eidx3 = edge_index.reshape(2, _NBR, 128)
    vals2 = edge_vals.reshape(_NBR, 128)

    t1 = pl.pallas_call(
        _mm1_body,
        out_shape=jax.ShapeDtypeStruct((_N, _H), jnp.float32),
    )(x, W1)

    p1 = _spmm1(t1, eidx3, vals2)

    t2 = pl.pallas_call(
        _mid_body,
        out_shape=jax.ShapeDtypeStruct((_NPAD, 1), jnp.float32),
    )(p1, b1.reshape(1, _H), W2)

    p2 = _spmm2(t2.reshape(_NPAD), eidx3, vals2)

    outp = pl.pallas_call(
        _fin_body,
        out_shape=jax.ShapeDtypeStruct((80, 128), jnp.float32),
    )(p2.reshape(_NC * _NS, 80, 128), b2.reshape(1, 1))

    return outp.reshape(_NPAD)[:_N].reshape(_N, 1)


# speedup vs baseline: 1.3553x; 1.1241x over previous
import functools

import jax
import jax.numpy as jnp
from jax import lax
from jax.experimental import pallas as pl
from jax.experimental.pallas import tpu as pltpu
from jax.experimental.pallas import tpu_sc as plsc

_N = 10000
_NPAD = 10240
_E = 320000
_H = 16
_NBR = 2500
_BPW = 80
_NC = 2
_NS = 16

_mesh = plsc.VectorSubcoreMesh(core_axis_name="c", subcore_axis_name="s")


def _load_edges_start(eidx_hbm, vals_hbm, src_v, dst_v, vals_v, wid, sems):
    base = wid * 78 + jnp.minimum(wid, 4)
    pltpu.async_copy(eidx_hbm.at[1, pl.ds(base, 78)],
                     src_v.at[pl.ds(0, 78)], sems[0])
    pltpu.async_copy(eidx_hbm.at[0, pl.ds(base, 78)],
                     dst_v.at[pl.ds(0, 78)], sems[1])
    pltpu.async_copy(vals_hbm.at[pl.ds(base, 78)],
                     vals_v.at[pl.ds(0, 78)], sems[2])

    @pl.when(wid < 4)
    def _extra():
        pltpu.async_copy(eidx_hbm.at[1, base + 78], src_v.at[78], sems[0])
        pltpu.async_copy(eidx_hbm.at[0, base + 78], dst_v.at[78], sems[1])
        pltpu.async_copy(vals_hbm.at[base + 78], vals_v.at[78], sems[2])

    zi16 = jnp.zeros((16,), jnp.int32)
    zf16 = jnp.zeros((16,), jnp.float32)
    for c in range(8):
        sl = pl.ds(c * 16, 16)
        src_v[79, sl] = zi16
        dst_v[79, sl] = zi16
        vals_v[79, sl] = zf16

    @pl.when(wid >= 4)
    def _z78():
        for c in range(8):
            sl = pl.ds(c * 16, 16)
            src_v[78, sl] = zi16
            dst_v[78, sl] = zi16
            vals_v[78, sl] = zf16
    return base


def _load_edges_wait(eidx_hbm, vals_hbm, src_v, dst_v, vals_v, wid, sems,
                     base):
    pltpu.make_async_copy(eidx_hbm.at[1, pl.ds(base, 78)],
                          src_v.at[pl.ds(0, 78)], sems[0]).wait()
    pltpu.make_async_copy(eidx_hbm.at[0, pl.ds(base, 78)],
                          dst_v.at[pl.ds(0, 78)], sems[1]).wait()
    pltpu.make_async_copy(vals_hbm.at[pl.ds(base, 78)],
                          vals_v.at[pl.ds(0, 78)], sems[2]).wait()

    @pl.when(wid < 4)
    def _extra():
        pltpu.make_async_copy(eidx_hbm.at[1, base + 78],
                              src_v.at[78], sems[0]).wait()
        pltpu.make_async_copy(eidx_hbm.at[0, base + 78],
                              dst_v.at[78], sems[1]).wait()
        pltpu.make_async_copy(vals_hbm.at[base + 78],
                              vals_v.at[78], sems[2]).wait()


def _mm1_body(x_ref, w_ref, o_ref):
    o_ref[...] = jnp.dot(x_ref[...], w_ref[...],
                         preferred_element_type=jnp.float32)


def _mid_body(p_ref, b1_ref, w2_ref, o_ref):
    m = p_ref[0] + p_ref[1]
    h = jnp.maximum(m + b1_ref[...], 0.0)
    o_ref[...] = jnp.dot(h, w2_ref[...], preferred_element_type=jnp.float32)


def _fin_body(p_ref, b2_ref, o_ref):
    o_ref[...] = jax.nn.sigmoid(jnp.sum(p_ref[...], axis=0) + b2_ref[...])


@functools.partial(
    pl.kernel,
    out_type=jax.ShapeDtypeStruct((_NC, _NPAD, _H), jnp.float32),
    mesh=_mesh,
    scratch_types=[
        pltpu.VMEM((_BPW, 128), jnp.int32),
        pltpu.VMEM((_BPW, 128), jnp.int32),
        pltpu.VMEM((_BPW, 128), jnp.float32),
        pltpu.VMEM((128, _H), jnp.float32),
        pltpu.VMEM((128, _H), jnp.float32),
        pltpu.VMEM((128, _H), jnp.float32),
        pltpu.VMEM((128, _H), jnp.float32),
        pltpu.VMEM((128, _H), jnp.float32),
        pltpu.VMEM((128, _H), jnp.float32),
        pltpu.VMEM((128, _H), jnp.float32),
        pltpu.VMEM((128, _H), jnp.float32),
        pltpu.VMEM((640, _H), jnp.float32),
        pltpu.VMEM_SHARED((_NPAD, _H), jnp.float32),
        pltpu.VMEM_SHARED((_NPAD, _H), jnp.float32),
        pltpu.SemaphoreType.DMA,
        pltpu.SemaphoreType.DMA,
        pltpu.SemaphoreType.DMA,
        pltpu.SemaphoreType.DMA,
        pltpu.SemaphoreType.DMA,
        pltpu.SemaphoreType.DMA,
        pltpu.SemaphoreType.DMA,
        pltpu.SemaphoreType.DMA,
        pltpu.SemaphoreType.DMA,
        pltpu.SemaphoreType.DMA,
        pltpu.SemaphoreType.DMA,
        pltpu.SemaphoreType.DMA,
        pltpu.SemaphoreType.DMA,
        pltpu.SemaphoreType.DMA,
        pltpu.SemaphoreType.DMA,
        pltpu.SemaphoreType.DMA,
    ],
    compiler_params=pltpu.CompilerParams(use_tc_tiling_on_sc=False,
                                         needs_layout_passes=False),
)
def _spmm1(t1_hbm, eidx_hbm, vals_hbm, out_hbm,
           src_v, dst_v, vals_v, r0, r1, r2, r3, r4, r5, r6, r7, zeros_v,
           acc_sh, t1_sh, g0, g1, g2, g3, g4, g5, g6, g7,
           s0, s1, s2, s3, s4, s5, s6, s7):
    rows = (r0, r1, r2, r3, r4, r5, r6, r7)
    gsems = (g0, g1, g2, g3, g4, g5, g6, g7)
    ssems = (s0, s1, s2, s3, s4, s5, s6, s7)
    cid = lax.axis_index("c")
    sid = lax.axis_index("s")
    wid = sid * _NC + cid

    pltpu.async_copy(t1_hbm.at[pl.ds(sid * 625, 625)],
                     t1_sh.at[pl.ds(sid * 625, 625)], s3)
    ebase = _load_edges_start(eidx_hbm, vals_hbm, src_v, dst_v, vals_v,
                              wid, (s0, s1, s2))

    zv = jnp.zeros((_H,), jnp.float32)

    @pl.loop(0, 640, unroll=8)
    def _zero(i):
        zeros_v[i, :] = zv

    pltpu.sync_copy(zeros_v, acc_sh.at[pl.ds(sid * 640, 640)])
    pltpu.make_async_copy(t1_hbm.at[pl.ds(sid * 625, 625)],
                          t1_sh.at[pl.ds(sid * 625, 625)], s3).wait()
    _load_edges_wait(eidx_hbm, vals_hbm, src_v, dst_v, vals_v, wid,
                     (s0, s1, s2), ebase)

    plsc.subcore_barrier()

    def _compute(rbuf, b):
        for g in range(8):
            vv = vals_v[b, pl.ds(g * 16, 16)]
            for j in range(16):
                e = g * 16 + j
                bj = jnp.broadcast_to(vv[j], (16,))
                rbuf[e, :] = rbuf[e, :] * bj

    for k in range(8):
        pltpu.async_copy(t1_sh.at[src_v.at[k]], rows[k], gsems[k])

    @pl.loop(0, 10)
    def _oct(q):
        b0 = q * 8
        for k in range(8):
            b = b0 + k
            pltpu.make_async_copy(t1_hbm.at[src_v.at[b]],
                                  rows[k], gsems[k]).wait()
            _compute(rows[k], b)
            pltpu.async_copy(rows[k], acc_sh.at[dst_v.at[b]], ssems[k],
                             add=True)

        @pl.when(q < 9)
        def _prefetch():
            for k in range(8):
                bn = b0 + 8 + k
                pltpu.make_async_copy(rows[k], acc_sh.at[dst_v.at[bn]],
                                      ssems[k]).wait()
                pltpu.async_copy(t1_sh.at[src_v.at[bn]], rows[k], gsems[k])

    for k in range(8):
        pltpu.make_async_copy(rows[k], acc_sh.at[dst_v.at[72 + k]],
                              ssems[k]).wait()

    plsc.subcore_barrier()
    pltpu.sync_copy(acc_sh.at[pl.ds(sid * 640, 640)],
                    out_hbm.at[cid, pl.ds(sid * 640, 640)])


@functools.partial(
    pl.kernel,
    out_type=jax.ShapeDtypeStruct((_NC, _NS, 640, _H), jnp.float32),
    mesh=_mesh,
    scratch_types=[
        pltpu.VMEM((_NPAD,), jnp.float32),
        pltpu.VMEM((640, _H), jnp.float32),
        pltpu.VMEM((_BPW, 128), jnp.int32),
        pltpu.VMEM((_BPW, 128), jnp.int32),
        pltpu.VMEM((_BPW, 128), jnp.float32),
        pltpu.SemaphoreType.DMA,
        pltpu.SemaphoreType.DMA,
        pltpu.SemaphoreType.DMA,
        pltpu.SemaphoreType.DMA,
    ],
    compiler_params=pltpu.CompilerParams(use_tc_tiling_on_sc=False,
                                         needs_layout_passes=False),
)
def _spmm2(t2_hbm, eidx_hbm, vals_hbm, out_hbm,
           t2_v, acc_v, src_v, dst_v, vals_v, m0, m1, m2, m3):
    cid = lax.axis_index("c")
    sid = lax.axis_index("s")
    wid = sid * _NC + cid

    pltpu.async_copy(t2_hbm, t2_v, m3)
    ebase = _load_edges_start(eidx_hbm, vals_hbm, src_v, dst_v, vals_v,
                              wid, (m0, m1, m2))

    zv = jnp.zeros((_H,), jnp.float32)

    @pl.loop(0, 640, unroll=8)
    def _zero(i):
        acc_v[i, :] = zv

    pltpu.make_async_copy(t2_hbm, t2_v, m3).wait()
    _load_edges_wait(eidx_hbm, vals_hbm, src_v, dst_v, vals_v, wid,
                     (m0, m1, m2), ebase)

    @pl.loop(0, _BPW)
    def _batch(b):
        for g in range(8):
            sl = pl.ds(g * 16, 16)
            sidx = src_v[b, sl]
            didx = dst_v[b, sl]
            vv = vals_v[b, sl]
            gathered = plsc.load_gather(t2_v, [sidx])
            contrib = gathered * vv
            plsc.addupdate_scatter(acc_v, [didx >> 4, didx & 15], contrib)

    pltpu.sync_copy(acc_v, out_hbm.at[cid, sid])


def kernel(x, edge_index, edge_vals, W1, b1, W2, b2):
    eidx3 = edge_index.reshape(2, _NBR, 128)
    vals2 = edge_vals.reshape(_NBR, 128)

    t1 = pl.pallas_call(
        _mm1_body,
        out_shape=jax.ShapeDtypeStruct((_N, _H), jnp.float32),
    )(x, W1)

    p1 = _spmm1(t1, eidx3, vals2)

    t2 = pl.pallas_call(
        _mid_body,
        out_shape=jax.ShapeDtypeStruct((_NPAD, 1), jnp.float32),
    )(p1, b1.reshape(1, _H), W2)

    p2 = _spmm2(t2.reshape(_NPAD), eidx3, vals2)

    outp = pl.pallas_call(
        _fin_body,
        out_shape=jax.ShapeDtypeStruct((80, 128), jnp.float32),
    )(p2.reshape(_NC * _NS, 80, 128), b2.reshape(1, 1))

    return outp.reshape(_NPAD)[:_N].reshape(_N, 1)


# speedup vs baseline: 1.3959x; 1.0300x over previous
import functools

import jax
import jax.numpy as jnp
from jax import lax
from jax.experimental import pallas as pl
from jax.experimental.pallas import tpu as pltpu
from jax.experimental.pallas import tpu_sc as plsc

_N = 10000
_NPAD = 10240
_E = 320000
_H = 16
_NBR = 2500
_BPW = 80
_NC = 2
_NS = 16

_mesh = plsc.VectorSubcoreMesh(core_axis_name="c", subcore_axis_name="s")


def _load_edges_start(eidx_hbm, vals_hbm, src_v, dst_v, vals_v, wid, sems):
    base = wid * 78 + jnp.minimum(wid, 4)
    pltpu.async_copy(eidx_hbm.at[1, pl.ds(base, 78)],
                     src_v.at[pl.ds(0, 78)], sems[0])
    pltpu.async_copy(eidx_hbm.at[0, pl.ds(base, 78)],
                     dst_v.at[pl.ds(0, 78)], sems[1])
    pltpu.async_copy(vals_hbm.at[pl.ds(base, 78)],
                     vals_v.at[pl.ds(0, 78)], sems[2])

    @pl.when(wid < 4)
    def _extra():
        pltpu.async_copy(eidx_hbm.at[1, base + 78], src_v.at[78], sems[0])
        pltpu.async_copy(eidx_hbm.at[0, base + 78], dst_v.at[78], sems[1])
        pltpu.async_copy(vals_hbm.at[base + 78], vals_v.at[78], sems[2])

    zi16 = jnp.zeros((16,), jnp.int32)
    zf16 = jnp.zeros((16,), jnp.float32)
    for c in range(8):
        sl = pl.ds(c * 16, 16)
        src_v[79, sl] = zi16
        dst_v[79, sl] = zi16
        vals_v[79, sl] = zf16

    @pl.when(wid >= 4)
    def _z78():
        for c in range(8):
            sl = pl.ds(c * 16, 16)
            src_v[78, sl] = zi16
            dst_v[78, sl] = zi16
            vals_v[78, sl] = zf16
    return base


def _load_edges_wait(eidx_hbm, vals_hbm, src_v, dst_v, vals_v, wid, sems,
                     base):
    pltpu.make_async_copy(eidx_hbm.at[1, pl.ds(base, 78)],
                          src_v.at[pl.ds(0, 78)], sems[0]).wait()
    pltpu.make_async_copy(eidx_hbm.at[0, pl.ds(base, 78)],
                          dst_v.at[pl.ds(0, 78)], sems[1]).wait()
    pltpu.make_async_copy(vals_hbm.at[pl.ds(base, 78)],
                          vals_v.at[pl.ds(0, 78)], sems[2]).wait()

    @pl.when(wid < 4)
    def _extra():
        pltpu.make_async_copy(eidx_hbm.at[1, base + 78],
                              src_v.at[78], sems[0]).wait()
        pltpu.make_async_copy(eidx_hbm.at[0, base + 78],
                              dst_v.at[78], sems[1]).wait()
        pltpu.make_async_copy(vals_hbm.at[base + 78],
                              vals_v.at[78], sems[2]).wait()


def _mm1_body(x_ref, w_ref, o_ref):
    o_ref[...] = jnp.dot(x_ref[...], w_ref[...],
                         preferred_element_type=jnp.float32)


def _mid_body(p_ref, b1_ref, w2_ref, o_ref):
    m = p_ref[0] + p_ref[1]
    h = jnp.maximum(m + b1_ref[...], 0.0)
    o_ref[...] = jnp.dot(h, w2_ref[...], preferred_element_type=jnp.float32)


def _fin_body(p_ref, b2_ref, o_ref):
    o_ref[...] = jax.nn.sigmoid(jnp.sum(p_ref[...], axis=0) + b2_ref[...])


@functools.partial(
    pl.kernel,
    out_type=jax.ShapeDtypeStruct((_NC, _NPAD, _H), jnp.float32),
    mesh=_mesh,
    scratch_types=[
        pltpu.VMEM((_BPW, 128), jnp.int32),
        pltpu.VMEM((_BPW, 128), jnp.int32),
        pltpu.VMEM((_BPW, 128), jnp.float32),
        pltpu.VMEM((128, _H), jnp.float32),
        pltpu.VMEM((128, _H), jnp.float32),
        pltpu.VMEM((128, _H), jnp.float32),
        pltpu.VMEM((128, _H), jnp.float32),
        pltpu.VMEM((640, _H), jnp.float32),
        pltpu.VMEM_SHARED((_NPAD, _H), jnp.float32),
        pltpu.VMEM_SHARED((_NPAD, _H), jnp.float32),
        pltpu.SemaphoreType.DMA,
        pltpu.SemaphoreType.DMA,
        pltpu.SemaphoreType.DMA,
        pltpu.SemaphoreType.DMA,
        pltpu.SemaphoreType.DMA,
        pltpu.SemaphoreType.DMA,
        pltpu.SemaphoreType.DMA,
        pltpu.SemaphoreType.DMA,
    ],
    compiler_params=pltpu.CompilerParams(use_tc_tiling_on_sc=False,
                                         needs_layout_passes=False),
)
def _spmm1(t1_hbm, eidx_hbm, vals_hbm, out_hbm,
           src_v, dst_v, vals_v, r0, r1, r2, r3, zeros_v, acc_sh, t1_sh,
           g0, g1, g2, g3, s0, s1, s2, s3):
    rows = (r0, r1, r2, r3)
    gsems = (g0, g1, g2, g3)
    ssems = (s0, s1, s2, s3)
    cid = lax.axis_index("c")
    sid = lax.axis_index("s")
    wid = sid * _NC + cid

    pltpu.async_copy(t1_hbm.at[pl.ds(sid * 625, 625)],
                     t1_sh.at[pl.ds(sid * 625, 625)], s3)
    ebase = _load_edges_start(eidx_hbm, vals_hbm, src_v, dst_v, vals_v,
                              wid, (s0, s1, s2))

    zv = jnp.zeros((_H,), jnp.float32)

    @pl.loop(0, 640, unroll=8)
    def _zero(i):
        zeros_v[i, :] = zv

    pltpu.sync_copy(zeros_v, acc_sh.at[pl.ds(sid * 640, 640)])
    pltpu.make_async_copy(t1_hbm.at[pl.ds(sid * 625, 625)],
                          t1_sh.at[pl.ds(sid * 625, 625)], s3).wait()
    _load_edges_wait(eidx_hbm, vals_hbm, src_v, dst_v, vals_v, wid,
                     (s0, s1, s2), ebase)

    plsc.subcore_barrier()

    def _compute(rbuf, b):
        for g in range(8):
            vv = vals_v[b, pl.ds(g * 16, 16)]
            for j in range(16):
                e = g * 16 + j
                bj = jnp.broadcast_to(vv[j], (16,))
                rbuf[e, :] = rbuf[e, :] * bj

    for k in range(4):
        pltpu.async_copy(t1_sh.at[src_v.at[k]], rows[k], gsems[k])

    @pl.loop(0, 20)
    def _quad(q):
        b0 = q * 4
        for k in range(4):
            b = b0 + k
            pltpu.make_async_copy(t1_hbm.at[src_v.at[b]],
                                  rows[k], gsems[k]).wait()
            _compute(rows[k], b)
            pltpu.async_copy(rows[k], acc_sh.at[dst_v.at[b]], ssems[k],
                             add=True)

        @pl.when(q < 19)
        def _prefetch():
            for k in range(4):
                bn = b0 + 4 + k
                pltpu.make_async_copy(rows[k], acc_sh.at[dst_v.at[bn]],
                                      ssems[k]).wait()
                pltpu.async_copy(t1_sh.at[src_v.at[bn]], rows[k], gsems[k])

    for k in range(4):
        pltpu.make_async_copy(rows[k], acc_sh.at[dst_v.at[76 + k]],
                              ssems[k]).wait()

    plsc.subcore_barrier()
    pltpu.sync_copy(acc_sh.at[pl.ds(sid * 640, 640)],
                    out_hbm.at[cid, pl.ds(sid * 640, 640)])


@functools.partial(
    pl.kernel,
    out_type=jax.ShapeDtypeStruct((_NC, _NS, 640, _H), jnp.float32),
    mesh=_mesh,
    scratch_types=[
        pltpu.VMEM((_NPAD,), jnp.float32),
        pltpu.VMEM((640, _H), jnp.float32),
        pltpu.VMEM((_BPW, 128), jnp.int32),
        pltpu.VMEM((_BPW, 128), jnp.int32),
        pltpu.VMEM((_BPW, 128), jnp.float32),
        pltpu.SemaphoreType.DMA,
        pltpu.SemaphoreType.DMA,
        pltpu.SemaphoreType.DMA,
        pltpu.SemaphoreType.DMA,
    ],
    compiler_params=pltpu.CompilerParams(use_tc_tiling_on_sc=False,
                                         needs_layout_passes=False),
)
def _spmm2(t2_hbm, eidx_hbm, vals_hbm, out_hbm,
           t2_v, acc_v, src_v, dst_v, vals_v, m0, m1, m2, m3):
    cid = lax.axis_index("c")
    sid = lax.axis_index("s")
    wid = sid * _NC + cid

    pltpu.async_copy(t2_hbm, t2_v, m3)
    ebase = _load_edges_start(eidx_hbm, vals_hbm, src_v, dst_v, vals_v,
                              wid, (m0, m1, m2))

    zv = jnp.zeros((_H,), jnp.float32)

    @pl.loop(0, 640, unroll=8)
    def _zero(i):
        acc_v[i, :] = zv

    pltpu.make_async_copy(t2_hbm, t2_v, m3).wait()
    _load_edges_wait(eidx_hbm, vals_hbm, src_v, dst_v, vals_v, wid,
                     (m0, m1, m2), ebase)

    @pl.loop(0, _BPW)
    def _batch(b):
        for g in range(8):
            sl = pl.ds(g * 16, 16)
            sidx = src_v[b, sl]
            didx = dst_v[b, sl]
            vv = vals_v[b, sl]
            gathered = plsc.load_gather(t2_v, [sidx])
            contrib = gathered * vv
            plsc.addupdate_scatter(acc_v, [didx >> 4, didx & 15], contrib)

    pltpu.sync_copy(acc_v, out_hbm.at[cid, sid])


def kernel(x, edge_index, edge_vals, W1, b1, W2, b2):
    eidx3 = edge_index.reshape(2, _NBR, 128)
    vals2 = edge_vals.reshape(_NBR, 128)

    t1 = pl.pallas_call(
        _mm1_body,
        out_shape=jax.ShapeDtypeStruct((_N, _H), jnp.float32),
    )(x, W1)

    p1 = _spmm1(t1, eidx3, vals2)

    t2 = pl.pallas_call(
        _mid_body,
        out_shape=jax.ShapeDtypeStruct((_NPAD, 1), jnp.float32),
    )(p1, b1.reshape(1, _H), W2)

    p2 = _spmm2(t2.reshape(_NPAD), eidx3, vals2)

    outp = pl.pallas_call(
        _fin_body,
        out_shape=jax.ShapeDtypeStruct((80, 128), jnp.float32),
    )(p2.reshape(_NC * _NS, 80, 128), b2.reshape(1, 1))

    return outp.reshape(_NPAD)[:_N].reshape(_N, 1)


# speedup vs baseline: 1.5497x; 1.1101x over previous
import functools

import jax
import jax.numpy as jnp
from jax import lax
from jax.experimental import pallas as pl
from jax.experimental.pallas import tpu as pltpu
from jax.experimental.pallas import tpu_sc as plsc

_N = 10000
_NPAD = 10240
_E = 320000
_H = 16
_NBR = 2500
_BPW = 80
_NC = 2
_NS = 16

_mesh = plsc.VectorSubcoreMesh(core_axis_name="c", subcore_axis_name="s")


def _load_edges_start(eidx_hbm, vals_hbm, src_v, dst_v, vals_v, wid, sems):
    base = wid * 78 + jnp.minimum(wid, 4)
    pltpu.async_copy(eidx_hbm.at[1, pl.ds(base, 78)],
                     src_v.at[pl.ds(0, 78)], sems[0])
    pltpu.async_copy(eidx_hbm.at[0, pl.ds(base, 78)],
                     dst_v.at[pl.ds(0, 78)], sems[1])
    pltpu.async_copy(vals_hbm.at[pl.ds(base, 78)],
                     vals_v.at[pl.ds(0, 78)], sems[2])

    @pl.when(wid < 4)
    def _extra():
        pltpu.async_copy(eidx_hbm.at[1, base + 78], src_v.at[78], sems[0])
        pltpu.async_copy(eidx_hbm.at[0, base + 78], dst_v.at[78], sems[1])
        pltpu.async_copy(vals_hbm.at[base + 78], vals_v.at[78], sems[2])

    zi16 = jnp.zeros((16,), jnp.int32)
    zf16 = jnp.zeros((16,), jnp.float32)
    for c in range(8):
        sl = pl.ds(c * 16, 16)
        src_v[79, sl] = zi16
        dst_v[79, sl] = zi16
        vals_v[79, sl] = zf16

    @pl.when(wid >= 4)
    def _z78():
        for c in range(8):
            sl = pl.ds(c * 16, 16)
            src_v[78, sl] = zi16
            dst_v[78, sl] = zi16
            vals_v[78, sl] = zf16
    return base


def _load_edges_wait(eidx_hbm, vals_hbm, src_v, dst_v, vals_v, wid, sems,
                     base):
    pltpu.make_async_copy(eidx_hbm.at[1, pl.ds(base, 78)],
                          src_v.at[pl.ds(0, 78)], sems[0]).wait()
    pltpu.make_async_copy(eidx_hbm.at[0, pl.ds(base, 78)],
                          dst_v.at[pl.ds(0, 78)], sems[1]).wait()
    pltpu.make_async_copy(vals_hbm.at[pl.ds(base, 78)],
                          vals_v.at[pl.ds(0, 78)], sems[2]).wait()

    @pl.when(wid < 4)
    def _extra():
        pltpu.make_async_copy(eidx_hbm.at[1, base + 78],
                              src_v.at[78], sems[0]).wait()
        pltpu.make_async_copy(eidx_hbm.at[0, base + 78],
                              dst_v.at[78], sems[1]).wait()
        pltpu.make_async_copy(vals_hbm.at[base + 78],
                              vals_v.at[78], sems[2]).wait()


def _mm1_body(x_ref, w_ref, o_ref):
    o_ref[...] = jnp.dot(x_ref[...], w_ref[...],
                         preferred_element_type=jnp.float32)


def _mid_body(p_ref, b1_ref, w2_ref, o_ref):
    m = p_ref[0] + p_ref[1]
    h = jnp.maximum(m + b1_ref[...], 0.0)
    o_ref[...] = jnp.dot(h, w2_ref[...], preferred_element_type=jnp.float32)


def _fin_body(p_ref, b2_ref, o_ref):
    o_ref[...] = jax.nn.sigmoid(jnp.sum(p_ref[...], axis=0) + b2_ref[...])


@functools.partial(
    pl.kernel,
    out_type=jax.ShapeDtypeStruct((_NC, _NPAD, _H), jnp.float32),
    mesh=_mesh,
    scratch_types=[
        pltpu.VMEM((_BPW, 128), jnp.int32),
        pltpu.VMEM((_BPW, 128), jnp.int32),
        pltpu.VMEM((_BPW, 128), jnp.float32),
        pltpu.VMEM((128, _H), jnp.float32),
        pltpu.VMEM((128, _H), jnp.float32),
        pltpu.VMEM((128, _H), jnp.float32),
        pltpu.VMEM((128, _H), jnp.float32),
        pltpu.VMEM((640, _H), jnp.float32),
        pltpu.VMEM_SHARED((_NPAD, _H), jnp.float32),
        pltpu.VMEM_SHARED((_NPAD, _H), jnp.float32),
        pltpu.SemaphoreType.DMA,
        pltpu.SemaphoreType.DMA,
        pltpu.SemaphoreType.DMA,
        pltpu.SemaphoreType.DMA,
        pltpu.SemaphoreType.DMA,
        pltpu.SemaphoreType.DMA,
        pltpu.SemaphoreType.DMA,
        pltpu.SemaphoreType.DMA,
    ],
    compiler_params=pltpu.CompilerParams(use_tc_tiling_on_sc=False,
                                         needs_layout_passes=False),
)
def _spmm1(t1_hbm, eidx_hbm, vals_hbm, out_hbm,
           src_v, dst_v, vals_v, r0, r1, r2, r3, zeros_v, acc_sh, t1_sh,
           g0, g1, g2, g3, s0, s1, s2, s3):
    rows = (r0, r1, r2, r3)
    gsems = (g0, g1, g2, g3)
    ssems = (s0, s1, s2, s3)
    cid = lax.axis_index("c")
    sid = lax.axis_index("s")
    wid = sid * _NC + cid

    pltpu.async_copy(t1_hbm.at[pl.ds(sid * 625, 625)],
                     t1_sh.at[pl.ds(sid * 625, 625)], s3)
    ebase = _load_edges_start(eidx_hbm, vals_hbm, src_v, dst_v, vals_v,
                              wid, (s0, s1, s2))

    zv = jnp.zeros((_H,), jnp.float32)

    @pl.loop(0, 640, unroll=8)
    def _zero(i):
        zeros_v[i, :] = zv

    pltpu.sync_copy(zeros_v, acc_sh.at[pl.ds(sid * 640, 640)])
    pltpu.make_async_copy(t1_hbm.at[pl.ds(sid * 625, 625)],
                          t1_sh.at[pl.ds(sid * 625, 625)], s3).wait()
    _load_edges_wait(eidx_hbm, vals_hbm, src_v, dst_v, vals_v, wid,
                     (s0, s1, s2), ebase)

    plsc.subcore_barrier()

    def _compute(rbuf, b):
        for g in range(8):
            vv = vals_v[b, pl.ds(g * 16, 16)]
            for j in range(16):
                e = g * 16 + j
                bj = jnp.broadcast_to(vv[j], (16,))
                rbuf[e, :] = rbuf[e, :] * bj

    for k in range(4):
        pltpu.async_copy(t1_sh.at[src_v.at[k]], rows[k], gsems[k])

    @pl.loop(0, 20)
    def _quad(q):
        b0 = q * 4
        for k in range(4):
            b = b0 + k
            pltpu.make_async_copy(t1_hbm.at[src_v.at[b]],
                                  rows[k], gsems[k]).wait()
            _compute(rows[k], b)
            pltpu.async_copy(rows[k], acc_sh.at[dst_v.at[b]], ssems[k],
                             add=True)

        @pl.when(q < 19)
        def _prefetch():
            for k in range(4):
                bn = b0 + 4 + k
                pltpu.make_async_copy(rows[k], acc_sh.at[dst_v.at[bn]],
                                      ssems[k]).wait()
                pltpu.async_copy(t1_sh.at[src_v.at[bn]], rows[k], gsems[k])

    for k in range(4):
        pltpu.make_async_copy(rows[k], acc_sh.at[dst_v.at[76 + k]],
                              ssems[k]).wait()

    plsc.subcore_barrier()
    pltpu.sync_copy(acc_sh.at[pl.ds(sid * 640, 640)],
                    out_hbm.at[cid, pl.ds(sid * 640, 640)])


@functools.partial(
    pl.kernel,
    out_type=jax.ShapeDtypeStruct((_NC, _NS, 640, _H), jnp.float32),
    mesh=_mesh,
    scratch_types=[
        pltpu.VMEM((_NPAD,), jnp.float32),
        pltpu.VMEM((640, _H), jnp.float32),
        pltpu.VMEM((_BPW, 128), jnp.int32),
        pltpu.VMEM((_BPW, 128), jnp.int32),
        pltpu.VMEM((_BPW, 128), jnp.float32),
        pltpu.VMEM((640, _H), jnp.float32),
        pltpu.VMEM((640, _H), jnp.float32),
        pltpu.VMEM((640,), jnp.float32),
        pltpu.VMEM((1, _H), jnp.float32),
        pltpu.VMEM((1, _H), jnp.float32),
        pltpu.VMEM_SHARED((_NPAD,), jnp.float32),
        pltpu.SemaphoreType.DMA,
        pltpu.SemaphoreType.DMA,
        pltpu.SemaphoreType.DMA,
        pltpu.SemaphoreType.DMA,
        pltpu.SemaphoreType.DMA,
    ],
    compiler_params=pltpu.CompilerParams(use_tc_tiling_on_sc=False,
                                         needs_layout_passes=False),
)
def _spmm2(p1_hbm, b1_hbm, w2_hbm, eidx_hbm, vals_hbm, out_hbm,
           t2_v, acc_v, src_v, dst_v, vals_v, p0_v, p1c_v, t2loc,
           b1_v, w2_v, t2_sh, m0, m1, m2, m3, m4):
    cid = lax.axis_index("c")
    sid = lax.axis_index("s")
    wid = sid * _NC + cid

    nid0 = sid * 640
    pltpu.async_copy(p1_hbm.at[0, pl.ds(nid0, 640)], p0_v, m3)
    pltpu.async_copy(p1_hbm.at[1, pl.ds(nid0, 640)], p1c_v, m4)
    pltpu.sync_copy(b1_hbm, b1_v)
    pltpu.sync_copy(w2_hbm, w2_v)
    ebase = _load_edges_start(eidx_hbm, vals_hbm, src_v, dst_v, vals_v,
                              wid, (m0, m1, m2))

    zv = jnp.zeros((_H,), jnp.float32)

    @pl.loop(0, 640, unroll=8)
    def _zero(i):
        acc_v[i, :] = zv

    pltpu.make_async_copy(p1_hbm.at[0, pl.ds(nid0, 640)], p0_v, m3).wait()
    pltpu.make_async_copy(p1_hbm.at[1, pl.ds(nid0, 640)], p1c_v, m4).wait()

    b1vec = b1_v[0, :]
    w2vec = w2_v[0, :]
    lane0 = lax.iota(jnp.int32, 16) == 0

    @pl.loop(0, 640, unroll=4)
    def _t2row(r):
        h = jnp.maximum(p0_v[r, :] + p1c_v[r, :] + b1vec, 0.0)
        s = jnp.sum(h * w2vec)
        plsc.store_scatter(t2loc, [jnp.full((16,), 0, jnp.int32) + r],
                           jnp.broadcast_to(s, (16,)), mask=lane0)

    pltpu.sync_copy(t2loc, t2_sh.at[pl.ds(nid0, 640)])
    plsc.subcore_barrier()
    pltpu.sync_copy(t2_sh, t2_v)
    _load_edges_wait(eidx_hbm, vals_hbm, src_v, dst_v, vals_v, wid,
                     (m0, m1, m2), ebase)

    @pl.loop(0, _BPW)
    def _batch(b):
        for g in range(8):
            sl = pl.ds(g * 16, 16)
            sidx = src_v[b, sl]
            didx = dst_v[b, sl]
            vv = vals_v[b, sl]
            gathered = plsc.load_gather(t2_v, [sidx])
            contrib = gathered * vv
            plsc.addupdate_scatter(acc_v, [didx >> 4, didx & 15], contrib)

    pltpu.sync_copy(acc_v, out_hbm.at[cid, sid])


def kernel(x, edge_index, edge_vals, W1, b1, W2, b2):
    eidx3 = edge_index.reshape(2, _NBR, 128)
    vals2 = edge_vals.reshape(_NBR, 128)

    t1 = pl.pallas_call(
        _mm1_body,
        out_shape=jax.ShapeDtypeStruct((_N, _H), jnp.float32),
    )(x, W1)

    p1 = _spmm1(t1, eidx3, vals2)

    p2 = _spmm2(p1, b1.reshape(1, _H), W2.reshape(1, _H), eidx3, vals2)

    outp = pl.pallas_call(
        _fin_body,
        out_shape=jax.ShapeDtypeStruct((80, 128), jnp.float32),
    )(p2.reshape(_NC * _NS, 80, 128), b2.reshape(1, 1))

    return outp.reshape(_NPAD)[:_N].reshape(_N, 1)
